# baseline (speedup 1.0000x reference)
import functools

import jax
import jax.numpy as jnp
from jax import lax
from jax.experimental import pallas as pl
from jax.experimental.pallas import tpu as pltpu
from jax.experimental.pallas import tpu_sc as plsc

N = 50000
H = 256
NC = 2
NS = 16
NW = NC * NS
CH = 80
NCHUNKS = N // CH
MAXC = 20


@functools.partial(
    pl.kernel,
    out_type=jax.ShapeDtypeStruct((N, H), jnp.float32),
    mesh=plsc.VectorSubcoreMesh(core_axis_name="c", subcore_axis_name="s"),
    scratch_types=[
        pltpu.VMEM((CH,), jnp.int32),
        pltpu.VMEM((CH,), jnp.int32),
        pltpu.VMEM((CH, H), jnp.float32),
        pltpu.VMEM((CH, H), jnp.float32),
        pltpu.SemaphoreType.DMA,
        pltpu.SemaphoreType.DMA,
        pltpu.SemaphoreType.DMA,
        pltpu.SemaphoreType.DMA,
    ],
)
def _sc_gather(table_hbm, idx_hbm, out_hbm,
               idx0, idx1, rb0, rb1, sem_g0, sem_g1, sem_s0, sem_s1):
    wid = lax.axis_index("s") * NC + lax.axis_index("c")
    n_w = jnp.where(wid < 17, 20, 19)
    s_w = jnp.where(wid < 17, 20 * wid, 340 + 19 * (wid - 17))

    pltpu.sync_copy(idx_hbm.at[pl.ds(s_w * CH, CH)], idx0)
    pltpu.async_copy(table_hbm.at[idx0], rb0, sem_g0)
    pltpu.sync_copy(idx_hbm.at[pl.ds((s_w + 1) * CH, CH)], idx1)
    pltpu.async_copy(table_hbm.at[idx1], rb1, sem_g1)

    def body(p, carry):
        a = 2 * p
        b = a + 1

        pltpu.make_async_copy(table_hbm.at[idx0], rb0, sem_g0).wait()
        pltpu.async_copy(rb0, out_hbm.at[pl.ds((s_w + a) * CH, CH)], sem_s0)

        pltpu.make_async_copy(table_hbm.at[idx1], rb1, sem_g1).wait()

        @pl.when(b < n_w)
        def _():
            pltpu.async_copy(rb1, out_hbm.at[pl.ds((s_w + b) * CH, CH)], sem_s1)

        pltpu.make_async_copy(
            rb0, out_hbm.at[pl.ds((s_w + a) * CH, CH)], sem_s0).wait()

        @pl.when(a + 2 < MAXC)
        def _():
            pltpu.sync_copy(idx_hbm.at[pl.ds((s_w + a + 2) * CH, CH)], idx0)
            pltpu.async_copy(table_hbm.at[idx0], rb0, sem_g0)

        @pl.when(b < n_w)
        def _():
            pltpu.make_async_copy(
                rb1, out_hbm.at[pl.ds((s_w + b) * CH, CH)], sem_s1).wait()

        @pl.when(b + 2 < MAXC)
        def _():
            pltpu.sync_copy(
                idx_hbm.at[
                    pl.ds((s_w + jnp.minimum(b + 2, n_w - 1)) * CH, CH)],
                idx1)
            pltpu.async_copy(table_hbm.at[idx1], rb1, sem_g1)

        return carry

    lax.fori_loop(0, MAXC // 2, body, 0)


def _tc_copy_body(src_ref, dst_ref, sem):
    copy = pltpu.make_async_copy(src_ref, dst_ref, sem)
    copy.start()
    copy.wait()


def _tc_copy(x):
    return pl.pallas_call(
        _tc_copy_body,
        out_shape=jax.ShapeDtypeStruct((N, H), jnp.float32),
        in_specs=[pl.BlockSpec(memory_space=pl.ANY)],
        out_specs=pl.BlockSpec(memory_space=pl.ANY),
        scratch_shapes=[pltpu.SemaphoreType.DMA],
    )(x)


def kernel(ids, layer_num, h_skip, hps, embed_weight):
    out = _sc_gather(embed_weight, ids.astype(jnp.int32))
    return (out, _tc_copy(h_skip))


# speedup vs baseline: 17.7199x; 17.7199x over previous
import functools

import jax
import jax.numpy as jnp
from jax import lax
from jax.experimental import pallas as pl
from jax.experimental.pallas import tpu as pltpu
from jax.experimental.pallas import tpu_sc as plsc

N = 50000
H = 256
NC = 2
NS = 16
NW = NC * NS
CH = 80
NCHUNKS = N // CH
MAXC = 20
NBUF = 4


@functools.partial(
    pl.kernel,
    out_type=jax.ShapeDtypeStruct((N, H), jnp.float32),
    mesh=plsc.VectorSubcoreMesh(core_axis_name="c", subcore_axis_name="s"),
    scratch_types=[
        pltpu.VMEM((MAXC * CH,), jnp.int32),
        pltpu.VMEM((NBUF, CH, H), jnp.float32),
        pltpu.SemaphoreType.DMA((NBUF,)),
        pltpu.SemaphoreType.DMA((NBUF,)),
    ],
)
def _sc_gather(table_hbm, idx_hbm, out_hbm, idx_all, rb, sem_g, sem_s):
    wid = lax.axis_index("s") * NC + lax.axis_index("c")
    n_w = jnp.where(wid < 17, 20, 19)
    s_w = jnp.where(wid < 17, 20 * wid, 340 + 19 * (wid - 17))

    pltpu.sync_copy(idx_hbm.at[pl.ds(s_w * CH, (MAXC - 1) * CH)],
                    idx_all.at[pl.ds(0, (MAXC - 1) * CH)])
    pltpu.sync_copy(
        idx_hbm.at[pl.ds((s_w + n_w - 1) * CH, CH)],
        idx_all.at[pl.ds((MAXC - 1) * CH, CH)])

    def gather(j, s):
        pltpu.async_copy(
            table_hbm.at[idx_all.at[pl.ds(j * CH, CH)]], rb.at[s],
            sem_g.at[s])

    def wait_gather(j, s):
        pltpu.make_async_copy(
            table_hbm.at[idx_all.at[pl.ds(j * CH, CH)]], rb.at[s],
            sem_g.at[s]).wait()

    def store(j, s):
        pltpu.async_copy(
            rb.at[s], out_hbm.at[pl.ds((s_w + j) * CH, CH)], sem_s.at[s])

    def wait_store(j, s):
        pltpu.make_async_copy(
            rb.at[s], out_hbm.at[pl.ds((s_w + j) * CH, CH)],
            sem_s.at[s]).wait()

    for s in range(NBUF):
        gather(s, s)

    def body(p, carry):
        for s in range(NBUF):
            j = NBUF * p + s
            wait_gather(j, s)

            @pl.when(j < n_w)
            def _():
                store(j, s)

        for s in range(NBUF):
            j = NBUF * p + s

            @pl.when(j < n_w)
            def _():
                wait_store(j, s)

            @pl.when(j + NBUF < MAXC)
            def _():
                gather(j + NBUF, s)

        return carry

    lax.fori_loop(0, MAXC // NBUF, body, 0)


_COPY_BLK = 2000


def _tc_copy_body(src_ref, dst_ref):
    dst_ref[...] = src_ref[...]


def _tc_copy(x):
    return pl.pallas_call(
        _tc_copy_body,
        out_shape=jax.ShapeDtypeStruct((N, H), jnp.float32),
        grid=(N // _COPY_BLK,),
        in_specs=[pl.BlockSpec((_COPY_BLK, H), lambda i: (i, 0))],
        out_specs=pl.BlockSpec((_COPY_BLK, H), lambda i: (i, 0)),
    )(x)


def kernel(ids, layer_num, h_skip, hps, embed_weight):
    out = _sc_gather(embed_weight, ids.astype(jnp.int32))
    return (out, _tc_copy(h_skip))


# speedup vs baseline: 17.8758x; 1.0088x over previous
import functools

import jax
import jax.numpy as jnp
from jax import lax
from jax.experimental import pallas as pl
from jax.experimental.pallas import tpu as pltpu
from jax.experimental.pallas import tpu_sc as plsc

N = 50000
H = 256
NC = 2
NS = 16
NW = NC * NS
CH = 80
NCHUNKS = N // CH
MAXC = 20
NBUF = 5


@functools.partial(
    pl.kernel,
    out_type=jax.ShapeDtypeStruct((N, H), jnp.float32),
    mesh=plsc.VectorSubcoreMesh(core_axis_name="c", subcore_axis_name="s"),
    scratch_types=[
        pltpu.VMEM((MAXC * CH,), jnp.int32),
        pltpu.VMEM((NBUF, CH, H), jnp.float32),
        pltpu.SemaphoreType.DMA((NBUF,)),
        pltpu.SemaphoreType.DMA((NBUF,)),
    ],
)
def _sc_gather(table_hbm, idx_hbm, out_hbm, idx_all, rb, sem_g, sem_s):
    wid = lax.axis_index("s") * NC + lax.axis_index("c")
    n_w = jnp.where(wid < 17, 20, 19)
    s_w = jnp.where(wid < 17, 20 * wid, 340 + 19 * (wid - 17))

    def gather(j, s):
        pltpu.async_copy(
            table_hbm.at[idx_all.at[pl.ds(j * CH, CH)]], rb.at[s],
            sem_g.at[s])

    def wait_gather(j, s):
        pltpu.make_async_copy(
            table_hbm.at[idx_all.at[pl.ds(j * CH, CH)]], rb.at[s],
            sem_g.at[s]).wait()

    def store(j, s):
        pltpu.async_copy(
            rb.at[s], out_hbm.at[pl.ds((s_w + j) * CH, CH)], sem_s.at[s])

    def wait_store(j, s):
        pltpu.make_async_copy(
            rb.at[s], out_hbm.at[pl.ds((s_w + j) * CH, CH)],
            sem_s.at[s]).wait()

    pltpu.sync_copy(idx_hbm.at[pl.ds(s_w * CH, NBUF * CH)],
                    idx_all.at[pl.ds(0, NBUF * CH)])
    for s in range(NBUF):
        gather(s, s)
    pltpu.sync_copy(
        idx_hbm.at[pl.ds((s_w + NBUF) * CH, (MAXC - 1 - NBUF) * CH)],
        idx_all.at[pl.ds(NBUF * CH, (MAXC - 1 - NBUF) * CH)])
    pltpu.sync_copy(
        idx_hbm.at[pl.ds((s_w + n_w - 1) * CH, CH)],
        idx_all.at[pl.ds((MAXC - 1) * CH, CH)])

    def body(p, carry):
        for s in range(NBUF):
            j = NBUF * p + s
            wait_gather(j, s)

            @pl.when(j < n_w)
            def _():
                store(j, s)

        for s in range(NBUF):
            j = NBUF * p + s

            @pl.when(j < n_w)
            def _():
                wait_store(j, s)

            @pl.when(j + NBUF < MAXC)
            def _():
                gather(j + NBUF, s)

        return carry

    lax.fori_loop(0, MAXC // NBUF, body, 0)


_COPY_BLK = 5000


def _tc_copy_body(src_ref, dst_ref):
    dst_ref[...] = src_ref[...]


def _tc_copy(x):
    return pl.pallas_call(
        _tc_copy_body,
        out_shape=jax.ShapeDtypeStruct((N, H), jnp.float32),
        grid=(N // _COPY_BLK,),
        in_specs=[pl.BlockSpec((_COPY_BLK, H), lambda i: (i, 0))],
        out_specs=pl.BlockSpec((_COPY_BLK, H), lambda i: (i, 0)),
    )(x)


def kernel(ids, layer_num, h_skip, hps, embed_weight):
    out = _sc_gather(embed_weight, ids.astype(jnp.int32))
    return (out, _tc_copy(h_skip))
